# Initial kernel scaffold; baseline (speedup 1.0000x reference)
import jax
import jax.numpy as jnp
from jax import lax
from jax.experimental import pallas as pl
from jax.experimental.pallas import tpu as pltpu
from jax.experimental.pallas import tpu_sc as plsc

N = 10000
E = 320000
D_IN = 128
D_HID = 256
D_OUT = 128

NC = 2
NS = 16
NW = NC * NS
CH = 128
NCH_W = 79
EP = NW * NCH_W * CH
PAD = EP - E
ROWS = EP // CH
NTRASH = 16
NPAD = N + NTRASH
ZR = NPAD // NS
OUTR = N // NS
CW = 8


def _seg_body(xp_hbm, src_hbm, dst_hbm, zeros_hbm, out_hbm,
              acc_sh, src_v, dst_v, rows_v, sem):
    c = lax.axis_index("c")
    s = lax.axis_index("s")
    wid = c * NS + s
    pltpu.sync_copy(zeros_hbm, acc_sh.at[pl.ds(s * ZR, ZR)])
    pltpu.sync_copy(src_hbm.at[pl.ds(wid * NCH_W, NCH_W)], src_v)
    pltpu.sync_copy(dst_hbm.at[pl.ds(wid * NCH_W, NCH_W)], dst_v)
    plsc.subcore_barrier()

    def body(j, carry):
        pltpu.async_copy(xp_hbm.at[src_v.at[j]], rows_v, sem).wait()
        pltpu.sync_copy(rows_v, acc_sh.at[dst_v.at[j]], add=True)
        return carry

    lax.fori_loop(0, NCH_W, body, 0)
    plsc.subcore_barrier()
    pltpu.sync_copy(acc_sh.at[pl.ds(s * OUTR, OUTR)], out_hbm.at[wid])


def _cnt_body(dst_hbm, zeros_hbm, ones_hbm, out_hbm,
              acc_sh, dst_v, ones_v):
    c = lax.axis_index("c")
    s = lax.axis_index("s")
    wid = c * NS + s
    pltpu.sync_copy(zeros_hbm, acc_sh.at[pl.ds(s * ZR, ZR)])
    pltpu.sync_copy(ones_hbm, ones_v)
    pltpu.sync_copy(dst_hbm.at[pl.ds(wid * NCH_W, NCH_W)], dst_v)
    plsc.subcore_barrier()

    def body(j, carry):
        pltpu.sync_copy(ones_v, acc_sh.at[dst_v.at[j]], add=True)
        return carry

    lax.fori_loop(0, NCH_W, body, 0)
    plsc.subcore_barrier()
    pltpu.sync_copy(acc_sh.at[pl.ds(s * OUTR, OUTR)], out_hbm.at[wid])


def _make_sc_calls():
    mesh = plsc.VectorSubcoreMesh(core_axis_name="c", subcore_axis_name="s")
    seg = pl.kernel(
        _seg_body,
        out_type=jax.ShapeDtypeStruct((NW, OUTR, D_IN), jnp.float32),
        mesh=mesh,
        scratch_types=[
            pltpu.VMEM_SHARED((NPAD, D_IN), jnp.float32),
            pltpu.VMEM((NCH_W, CH), jnp.int32),
            pltpu.VMEM((NCH_W, CH), jnp.int32),
            pltpu.VMEM((CH, D_IN), jnp.float32),
            pltpu.SemaphoreType.DMA,
        ],
    )
    cnt = pl.kernel(
        _cnt_body,
        out_type=jax.ShapeDtypeStruct((NW, OUTR, CW), jnp.float32),
        mesh=mesh,
        scratch_types=[
            pltpu.VMEM_SHARED((NPAD, CW), jnp.float32),
            pltpu.VMEM((NCH_W, CH), jnp.int32),
            pltpu.VMEM((CH, CW), jnp.float32),
        ],
    )
    return seg, cnt


def _indeg(cnt_ref):
    return cnt_ref[0, :, 0:1] + cnt_ref[1, :, 0:1]


def _prescale_body(x_ref, cnt_ref, xp_ref):
    deg = _indeg(cnt_ref) + 1.0
    dinv = lax.rsqrt(deg)
    xp_ref[...] = jnp.concatenate(
        [x_ref[...] * dinv, jnp.zeros((NTRASH, D_IN), jnp.float32)], axis=0)


def _dense_body(sp_ref, xp_ref, cnt_ref, w1_ref, b1_ref, w2_ref, h2_ref):
    indeg = _indeg(cnt_ref)
    deg = indeg + 1.0
    dinv = lax.rsqrt(deg)
    s = sp_ref[0] + sp_ref[1]
    xp = xp_ref[0:N, :]
    agg1 = (s + xp) * (dinv / deg)
    xi = lax.dot_general(agg1, w1_ref[...], (((1,), (1,)), ((), ())),
                         preferred_element_type=jnp.float32)
    xi = xi + b1_ref[...][None, :]
    xi = jnp.where(xi >= 0.0, xi, 0.01 * xi)
    mu = jnp.mean(xi, axis=0, keepdims=True)
    var = jnp.mean((xi - mu) ** 2, axis=0, keepdims=True)
    xin = (xi - mu) * lax.rsqrt(var + 1e-5)
    h2 = lax.dot_general(xin, w2_ref[...], (((1,), (1,)), ((), ())),
                         preferred_element_type=jnp.float32)
    h2_ref[...] = jnp.concatenate(
        [h2, jnp.zeros((NTRASH, D_OUT), jnp.float32)], axis=0)


def _final_body(s2_ref, cnt_ref, out_ref):
    indeg = _indeg(cnt_ref)
    inv2 = 1.0 / jnp.maximum(indeg, 1.0)
    out_ref[...] = jax.nn.sigmoid((s2_ref[0] + s2_ref[1]) * inv2)


def kernel(x, edge_index, W1, b1, W2):
    src = edge_index[0].astype(jnp.int32)
    dst = edge_index[1].astype(jnp.int32)
    pad_fill = N + (jnp.arange(PAD, dtype=jnp.int32) % NTRASH)
    src_p = jnp.concatenate([src, pad_fill]).reshape(ROWS, CH)
    dst_p = jnp.concatenate([dst, pad_fill]).reshape(ROWS, CH)
    zeros_d = jnp.zeros((ZR, D_IN), jnp.float32)
    zeros_c = jnp.zeros((ZR, CW), jnp.float32)
    ones_c = jnp.ones((CH, CW), jnp.float32)

    seg, cnt_call = _make_sc_calls()

    cnt = cnt_call(dst_p, zeros_c, ones_c).reshape(NC, N, CW)

    xp_pad = pl.pallas_call(
        _prescale_body,
        out_shape=jax.ShapeDtypeStruct((NPAD, D_IN), jnp.float32),
    )(x, cnt)

    sp = seg(xp_pad, src_p, dst_p, zeros_d).reshape(NC, N, D_IN)

    h2_pad = pl.pallas_call(
        _dense_body,
        out_shape=jax.ShapeDtypeStruct((NPAD, D_OUT), jnp.float32),
    )(sp, xp_pad, cnt, W1, b1, W2)

    s2 = seg(h2_pad, src_p, dst_p, zeros_d).reshape(NC, N, D_OUT)

    out = pl.pallas_call(
        _final_body,
        out_shape=jax.ShapeDtypeStruct((N, D_OUT), jnp.float32),
    )(s2, cnt)
    return out


# speedup vs baseline: 21.1496x; 21.1496x over previous
import jax
import jax.numpy as jnp
from jax import lax
from jax.experimental import pallas as pl
from jax.experimental.pallas import tpu as pltpu
from jax.experimental.pallas import tpu_sc as plsc

N = 10000
E = 320000
D_IN = 128
D_HID = 256
D_OUT = 128

NC = 2
NS = 16
NW = NC * NS
CH = 128
NCH_W = 80
EP = NW * NCH_W * CH
PAD = EP - E
ROWS = EP // CH
NTRASH = 112
NPAD = N + NTRASH
ZR = NPAD // NS
CW = 8


def _seg_body(xp_hbm, src_hbm, dst_hbm, zeros_hbm, out_hbm,
              acc_sh, src_v, dst_v, rows_v, sem):
    c = lax.axis_index("c")
    s = lax.axis_index("s")
    wid = c * NS + s
    pltpu.sync_copy(zeros_hbm, acc_sh.at[pl.ds(s * ZR, ZR)])
    pltpu.sync_copy(src_hbm.at[pl.ds(wid * NCH_W, NCH_W)], src_v)
    pltpu.sync_copy(dst_hbm.at[pl.ds(wid * NCH_W, NCH_W)], dst_v)
    plsc.subcore_barrier()

    def body(j, carry):
        pltpu.async_copy(xp_hbm.at[src_v.at[j]], rows_v, sem).wait()
        pltpu.sync_copy(rows_v, acc_sh.at[dst_v.at[j]], add=True)
        return carry

    lax.fori_loop(0, NCH_W, body, 0)
    plsc.subcore_barrier()
    pltpu.sync_copy(acc_sh.at[pl.ds(s * ZR, ZR)], out_hbm.at[wid])


def _cnt_body(dst_hbm, zeros_hbm, ones_hbm, out_hbm,
              acc_sh, dst_v, ones_v):
    c = lax.axis_index("c")
    s = lax.axis_index("s")
    wid = c * NS + s
    pltpu.sync_copy(zeros_hbm, acc_sh.at[pl.ds(s * ZR, ZR)])
    pltpu.sync_copy(ones_hbm, ones_v)
    pltpu.sync_copy(dst_hbm.at[pl.ds(wid * NCH_W, NCH_W)], dst_v)
    plsc.subcore_barrier()

    def body(j, carry):
        pltpu.sync_copy(ones_v, acc_sh.at[dst_v.at[j]], add=True)
        return carry

    lax.fori_loop(0, NCH_W, body, 0)
    plsc.subcore_barrier()
    pltpu.sync_copy(acc_sh.at[pl.ds(s * ZR, ZR)], out_hbm.at[wid])


def _make_sc_calls():
    mesh = plsc.VectorSubcoreMesh(core_axis_name="c", subcore_axis_name="s")
    seg = pl.kernel(
        _seg_body,
        out_type=jax.ShapeDtypeStruct((NW, ZR, D_IN), jnp.float32),
        mesh=mesh,
        scratch_types=[
            pltpu.VMEM_SHARED((NPAD, D_IN), jnp.float32),
            pltpu.VMEM((NCH_W, CH), jnp.int32),
            pltpu.VMEM((NCH_W, CH), jnp.int32),
            pltpu.VMEM((CH, D_IN), jnp.float32),
            pltpu.SemaphoreType.DMA,
        ],
    )
    cnt = pl.kernel(
        _cnt_body,
        out_type=jax.ShapeDtypeStruct((NW, ZR, CW), jnp.float32),
        mesh=mesh,
        scratch_types=[
            pltpu.VMEM_SHARED((NPAD, CW), jnp.float32),
            pltpu.VMEM((NCH_W, CH), jnp.int32),
            pltpu.VMEM((CH, CW), jnp.float32),
        ],
    )
    return seg, cnt


def _indeg(cnt_ref):
    return cnt_ref[0, 0:N, 0:1] + cnt_ref[1, 0:N, 0:1]


def _prescale_body(x_ref, cnt_ref, xp_ref):
    deg = _indeg(cnt_ref) + 1.0
    dinv = lax.rsqrt(deg)
    xp_ref[...] = jnp.concatenate(
        [x_ref[...] * dinv, jnp.zeros((NTRASH, D_IN), jnp.float32)], axis=0)


def _dense_body(sp_ref, xp_ref, cnt_ref, w1_ref, b1_ref, w2_ref, h2_ref):
    indeg = _indeg(cnt_ref)
    deg = indeg + 1.0
    dinv = lax.rsqrt(deg)
    s = sp_ref[0, 0:N, :] + sp_ref[1, 0:N, :]
    xp = xp_ref[0:N, :]
    agg1 = (s + xp) * (dinv / deg)
    xi = lax.dot_general(agg1, w1_ref[...], (((1,), (1,)), ((), ())),
                         preferred_element_type=jnp.float32)
    xi = xi + b1_ref[...][None, :]
    xi = jnp.where(xi >= 0.0, xi, 0.01 * xi)
    mu = jnp.mean(xi, axis=0, keepdims=True)
    var = jnp.mean((xi - mu) ** 2, axis=0, keepdims=True)
    xin = (xi - mu) * lax.rsqrt(var + 1e-5)
    h2 = lax.dot_general(xin, w2_ref[...], (((1,), (1,)), ((), ())),
                         preferred_element_type=jnp.float32)
    h2_ref[...] = jnp.concatenate(
        [h2, jnp.zeros((NTRASH, D_OUT), jnp.float32)], axis=0)


def _final_body(s2_ref, cnt_ref, out_ref):
    indeg = _indeg(cnt_ref)
    inv2 = 1.0 / jnp.maximum(indeg, 1.0)
    out_ref[...] = jax.nn.sigmoid((s2_ref[0, 0:N, :] + s2_ref[1, 0:N, :]) * inv2)


def kernel(x, edge_index, W1, b1, W2):
    src = edge_index[0].astype(jnp.int32)
    dst = edge_index[1].astype(jnp.int32)
    pad_fill = N + (jnp.arange(PAD, dtype=jnp.int32) % NTRASH)
    src_p = jnp.concatenate([src, pad_fill]).reshape(ROWS, CH)
    dst_p = jnp.concatenate([dst, pad_fill]).reshape(ROWS, CH)
    zeros_d = jnp.zeros((ZR, D_IN), jnp.float32)
    zeros_c = jnp.zeros((ZR, CW), jnp.float32)
    ones_c = jnp.ones((CH, CW), jnp.float32)

    seg, cnt_call = _make_sc_calls()

    cnt = cnt_call(dst_p, zeros_c, ones_c).reshape(NC, NPAD, CW)

    xp_pad = pl.pallas_call(
        _prescale_body,
        out_shape=jax.ShapeDtypeStruct((NPAD, D_IN), jnp.float32),
    )(x, cnt)

    sp = seg(xp_pad, src_p, dst_p, zeros_d).reshape(NC, NPAD, D_IN)

    h2_pad = pl.pallas_call(
        _dense_body,
        out_shape=jax.ShapeDtypeStruct((NPAD, D_OUT), jnp.float32),
    )(sp, xp_pad, cnt, W1, b1, W2)

    s2 = seg(h2_pad, src_p, dst_p, zeros_d).reshape(NC, NPAD, D_OUT)

    out = pl.pallas_call(
        _final_body,
        out_shape=jax.ShapeDtypeStruct((N, D_OUT), jnp.float32),
    )(s2, cnt)
    return out


# speedup vs baseline: 26.2800x; 1.2426x over previous
import jax
import jax.numpy as jnp
from jax import lax
from jax.experimental import pallas as pl
from jax.experimental.pallas import tpu as pltpu
from jax.experimental.pallas import tpu_sc as plsc

N = 10000
E = 320000
D_IN = 128
D_HID = 256
D_OUT = 128

NC = 2
NS = 16
NW = NC * NS
CH = 128
NCH_W = 80
EP = NW * NCH_W * CH
PAD = EP - E
ROWS = EP // CH
NTRASH = 112
NPAD = N + NTRASH
ZR = NPAD // NS
CW = 8


HB = NCH_W // 2


def _run_window(xp_hbm, src_hbm, dst_hbm, acc_sh,
                src_v, dst_v, rows_v, gsem0, gsem1, ssem, row0):
    pltpu.sync_copy(src_hbm.at[pl.ds(row0, HB)], src_v)
    pltpu.sync_copy(dst_hbm.at[pl.ds(row0, HB)], dst_v)

    pltpu.async_copy(xp_hbm.at[src_v.at[0]], rows_v.at[0], gsem0).wait()

    def body(g, carry):
        j = 2 * g
        gd1 = pltpu.async_copy(xp_hbm.at[src_v.at[j + 1]], rows_v.at[1], gsem1)
        sd0 = pltpu.async_copy(rows_v.at[0], acc_sh.at[dst_v.at[j]], ssem,
                               add=True)
        sd0.wait()
        gd1.wait()
        gd0 = pltpu.async_copy(xp_hbm.at[src_v.at[j + 2]], rows_v.at[0], gsem0)
        sd1 = pltpu.async_copy(rows_v.at[1], acc_sh.at[dst_v.at[j + 1]], ssem,
                               add=True)
        sd1.wait()
        gd0.wait()
        return carry

    lax.fori_loop(0, HB // 2 - 1, body, 0)
    gd1 = pltpu.async_copy(xp_hbm.at[src_v.at[HB - 1]], rows_v.at[1], gsem1)
    sd0 = pltpu.async_copy(rows_v.at[0], acc_sh.at[dst_v.at[HB - 2]], ssem,
                           add=True)
    sd0.wait()
    gd1.wait()
    pltpu.async_copy(rows_v.at[1], acc_sh.at[dst_v.at[HB - 1]], ssem,
                     add=True).wait()


def _seg_body(xp_hbm, src_hbm, dst_hbm, zeros_hbm, out_hbm,
              acc_sh, src_v, dst_v, rows_v, gsem0, gsem1, ssem):
    c = lax.axis_index("c")
    s = lax.axis_index("s")
    wid = c * NS + s
    pltpu.sync_copy(zeros_hbm, acc_sh.at[pl.ds(s * ZR, ZR)])
    plsc.subcore_barrier()
    _run_window(xp_hbm, src_hbm, dst_hbm, acc_sh,
                src_v, dst_v, rows_v, gsem0, gsem1, ssem, wid * NCH_W)
    _run_window(xp_hbm, src_hbm, dst_hbm, acc_sh,
                src_v, dst_v, rows_v, gsem0, gsem1, ssem, wid * NCH_W + HB)
    plsc.subcore_barrier()
    pltpu.sync_copy(acc_sh.at[pl.ds(s * ZR, ZR)], out_hbm.at[wid])


def _cnt_body(dst_hbm, zeros_hbm, ones_hbm, out_hbm,
              acc_sh, dst_v, ones_v):
    c = lax.axis_index("c")
    s = lax.axis_index("s")
    wid = c * NS + s
    pltpu.sync_copy(zeros_hbm, acc_sh.at[pl.ds(s * ZR, ZR)])
    pltpu.sync_copy(ones_hbm, ones_v)
    pltpu.sync_copy(dst_hbm.at[pl.ds(wid * NCH_W, NCH_W)], dst_v)
    plsc.subcore_barrier()

    def body(j, carry):
        pltpu.sync_copy(ones_v, acc_sh.at[dst_v.at[j]], add=True)
        return carry

    lax.fori_loop(0, NCH_W, body, 0)
    plsc.subcore_barrier()
    pltpu.sync_copy(acc_sh.at[pl.ds(s * ZR, ZR)], out_hbm.at[wid])


def _make_sc_calls():
    mesh = plsc.VectorSubcoreMesh(core_axis_name="c", subcore_axis_name="s")
    seg = pl.kernel(
        _seg_body,
        out_type=jax.ShapeDtypeStruct((NW, ZR, D_IN), jnp.float32),
        mesh=mesh,
        scratch_types=[
            pltpu.VMEM_SHARED((NPAD, D_IN), jnp.float32),
            pltpu.VMEM((HB, CH), jnp.int32),
            pltpu.VMEM((HB, CH), jnp.int32),
            pltpu.VMEM((2, CH, D_IN), jnp.float32),
            pltpu.SemaphoreType.DMA,
            pltpu.SemaphoreType.DMA,
            pltpu.SemaphoreType.DMA,
        ],
    )
    cnt = pl.kernel(
        _cnt_body,
        out_type=jax.ShapeDtypeStruct((NW, ZR, CW), jnp.float32),
        mesh=mesh,
        scratch_types=[
            pltpu.VMEM_SHARED((NPAD, CW), jnp.float32),
            pltpu.VMEM((NCH_W, CH), jnp.int32),
            pltpu.VMEM((CH, CW), jnp.float32),
        ],
    )
    return seg, cnt


def _indeg(cnt_ref):
    return cnt_ref[0, 0:N, 0:1] + cnt_ref[1, 0:N, 0:1]


def _prescale_body(x_ref, cnt_ref, xp_ref):
    deg = _indeg(cnt_ref) + 1.0
    dinv = lax.rsqrt(deg)
    xp_ref[...] = jnp.concatenate(
        [x_ref[...] * dinv, jnp.zeros((NTRASH, D_IN), jnp.float32)], axis=0)


def _dense_body(sp_ref, xp_ref, cnt_ref, w1_ref, b1_ref, w2_ref, h2_ref):
    indeg = _indeg(cnt_ref)
    deg = indeg + 1.0
    dinv = lax.rsqrt(deg)
    s = sp_ref[0, 0:N, :] + sp_ref[1, 0:N, :]
    xp = xp_ref[0:N, :]
    agg1 = (s + xp) * (dinv / deg)
    xi = lax.dot_general(agg1, w1_ref[...], (((1,), (1,)), ((), ())),
                         preferred_element_type=jnp.float32)
    xi = xi + b1_ref[...][None, :]
    xi = jnp.where(xi >= 0.0, xi, 0.01 * xi)
    mu = jnp.mean(xi, axis=0, keepdims=True)
    var = jnp.mean((xi - mu) ** 2, axis=0, keepdims=True)
    xin = (xi - mu) * lax.rsqrt(var + 1e-5)
    h2 = lax.dot_general(xin, w2_ref[...], (((1,), (1,)), ((), ())),
                         preferred_element_type=jnp.float32)
    h2_ref[...] = jnp.concatenate(
        [h2, jnp.zeros((NTRASH, D_OUT), jnp.float32)], axis=0)


def _final_body(s2_ref, cnt_ref, out_ref):
    indeg = _indeg(cnt_ref)
    inv2 = 1.0 / jnp.maximum(indeg, 1.0)
    out_ref[...] = jax.nn.sigmoid((s2_ref[0, 0:N, :] + s2_ref[1, 0:N, :]) * inv2)


def kernel(x, edge_index, W1, b1, W2):
    src = edge_index[0].astype(jnp.int32)
    dst = edge_index[1].astype(jnp.int32)
    pad_fill = N + (jnp.arange(PAD, dtype=jnp.int32) % NTRASH)
    src_p = jnp.concatenate([src, pad_fill]).reshape(ROWS, CH)
    dst_p = jnp.concatenate([dst, pad_fill]).reshape(ROWS, CH)
    zeros_d = jnp.zeros((ZR, D_IN), jnp.float32)
    zeros_c = jnp.zeros((ZR, CW), jnp.float32)
    ones_c = jnp.ones((CH, CW), jnp.float32)

    seg, cnt_call = _make_sc_calls()

    cnt = cnt_call(dst_p, zeros_c, ones_c).reshape(NC, NPAD, CW)

    xp_pad = pl.pallas_call(
        _prescale_body,
        out_shape=jax.ShapeDtypeStruct((NPAD, D_IN), jnp.float32),
    )(x, cnt)

    sp = seg(xp_pad, src_p, dst_p, zeros_d).reshape(NC, NPAD, D_IN)

    h2_pad = pl.pallas_call(
        _dense_body,
        out_shape=jax.ShapeDtypeStruct((NPAD, D_OUT), jnp.float32),
    )(sp, xp_pad, cnt, W1, b1, W2)

    s2 = seg(h2_pad, src_p, dst_p, zeros_d).reshape(NC, NPAD, D_OUT)

    out = pl.pallas_call(
        _final_body,
        out_shape=jax.ShapeDtypeStruct((N, D_OUT), jnp.float32),
    )(s2, cnt)
    return out


# speedup vs baseline: 29.7746x; 1.1330x over previous
import jax
import jax.numpy as jnp
from jax import lax
from jax.experimental import pallas as pl
from jax.experimental.pallas import tpu as pltpu
from jax.experimental.pallas import tpu_sc as plsc

N = 10000
E = 320000
D_IN = 128
D_HID = 256
D_OUT = 128

NC = 2
NS = 16
NW = NC * NS
CH = 128
NCH_W = 80
EP = NW * NCH_W * CH
PAD = EP - E
ROWS = EP // CH
NTRASH = 112
NPAD = N + NTRASH
ZR = NPAD // NS
CW = 8


HB = NCH_W // 2


def _run_window(xp_hbm, src_hbm, dst_hbm, acc_sh,
                src_v, dst_v, rows_v, gsem0, gsem1, ssem, row0):
    pltpu.sync_copy(src_hbm.at[pl.ds(row0, HB)], src_v)
    pltpu.sync_copy(dst_hbm.at[pl.ds(row0, HB)], dst_v)


    pltpu.async_copy(xp_hbm.at[src_v.at[0]], rows_v.at[0], gsem0)
    pltpu.async_copy(xp_hbm.at[src_v.at[1]], rows_v.at[1], gsem1)

    def body(g, carry):
        j = 2 * g
        pltpu.make_async_copy(xp_hbm.at[src_v.at[j]], rows_v.at[0],
                              gsem0).wait()
        pltpu.async_copy(rows_v.at[0], acc_sh.at[dst_v.at[j]], ssem,
                         add=True).wait()
        pltpu.async_copy(xp_hbm.at[src_v.at[j + 2]], rows_v.at[0], gsem0)
        pltpu.make_async_copy(xp_hbm.at[src_v.at[j + 1]], rows_v.at[1],
                              gsem1).wait()
        pltpu.async_copy(rows_v.at[1], acc_sh.at[dst_v.at[j + 1]], ssem,
                         add=True).wait()
        pltpu.async_copy(xp_hbm.at[src_v.at[j + 3]], rows_v.at[1], gsem1)
        return carry

    lax.fori_loop(0, HB // 2 - 1, body, 0)
    pltpu.make_async_copy(xp_hbm.at[src_v.at[HB - 2]], rows_v.at[0],
                          gsem0).wait()
    pltpu.async_copy(rows_v.at[0], acc_sh.at[dst_v.at[HB - 2]], ssem,
                     add=True).wait()
    pltpu.make_async_copy(xp_hbm.at[src_v.at[HB - 1]], rows_v.at[1],
                          gsem1).wait()
    pltpu.async_copy(rows_v.at[1], acc_sh.at[dst_v.at[HB - 1]], ssem,
                     add=True).wait()


def _seg_body(xp_hbm, src_hbm, dst_hbm, zeros_hbm, out_hbm,
              acc_sh, src_v, dst_v, rows_v, gsem0, gsem1, ssem):
    c = lax.axis_index("c")
    s = lax.axis_index("s")
    wid = c * NS + s
    pltpu.sync_copy(zeros_hbm, acc_sh.at[pl.ds(s * ZR, ZR)])
    plsc.subcore_barrier()
    _run_window(xp_hbm, src_hbm, dst_hbm, acc_sh,
                src_v, dst_v, rows_v, gsem0, gsem1, ssem, wid * NCH_W)
    _run_window(xp_hbm, src_hbm, dst_hbm, acc_sh,
                src_v, dst_v, rows_v, gsem0, gsem1, ssem, wid * NCH_W + HB)
    plsc.subcore_barrier()
    pltpu.sync_copy(acc_sh.at[pl.ds(s * ZR, ZR)], out_hbm.at[wid])


def _cnt_body(dst_hbm, zeros_hbm, ones_hbm, out_hbm,
              acc_sh, dst_v, ones_v):
    c = lax.axis_index("c")
    s = lax.axis_index("s")
    wid = c * NS + s
    pltpu.sync_copy(zeros_hbm, acc_sh.at[pl.ds(s * ZR, ZR)])
    pltpu.sync_copy(ones_hbm, ones_v)
    pltpu.sync_copy(dst_hbm.at[pl.ds(wid * NCH_W, NCH_W)], dst_v)
    plsc.subcore_barrier()

    def body(j, carry):
        pltpu.sync_copy(ones_v, acc_sh.at[dst_v.at[j]], add=True)
        return carry

    lax.fori_loop(0, NCH_W, body, 0)
    plsc.subcore_barrier()
    pltpu.sync_copy(acc_sh.at[pl.ds(s * ZR, ZR)], out_hbm.at[wid])


def _make_sc_calls():
    mesh = plsc.VectorSubcoreMesh(core_axis_name="c", subcore_axis_name="s")
    seg = pl.kernel(
        _seg_body,
        out_type=jax.ShapeDtypeStruct((NW, ZR, D_IN), jnp.float32),
        mesh=mesh,
        scratch_types=[
            pltpu.VMEM_SHARED((NPAD, D_IN), jnp.float32),
            pltpu.VMEM((HB, CH), jnp.int32),
            pltpu.VMEM((HB, CH), jnp.int32),
            pltpu.VMEM((2, CH, D_IN), jnp.float32),
            pltpu.SemaphoreType.DMA,
            pltpu.SemaphoreType.DMA,
            pltpu.SemaphoreType.DMA,
        ],
    )
    cnt = pl.kernel(
        _cnt_body,
        out_type=jax.ShapeDtypeStruct((NW, ZR, CW), jnp.float32),
        mesh=mesh,
        scratch_types=[
            pltpu.VMEM_SHARED((NPAD, CW), jnp.float32),
            pltpu.VMEM((NCH_W, CH), jnp.int32),
            pltpu.VMEM((CH, CW), jnp.float32),
        ],
    )
    return seg, cnt


def _indeg(cnt_ref):
    return cnt_ref[0, 0:N, 0:1] + cnt_ref[1, 0:N, 0:1]


def _prescale_body(x_ref, cnt_ref, xp_ref):
    deg = _indeg(cnt_ref) + 1.0
    dinv = lax.rsqrt(deg)
    xp_ref[...] = jnp.concatenate(
        [x_ref[...] * dinv, jnp.zeros((NTRASH, D_IN), jnp.float32)], axis=0)


def _dense_body(sp_ref, xp_ref, cnt_ref, w1_ref, b1_ref, w2_ref, h2_ref):
    indeg = _indeg(cnt_ref)
    deg = indeg + 1.0
    dinv = lax.rsqrt(deg)
    s = sp_ref[0, 0:N, :] + sp_ref[1, 0:N, :]
    xp = xp_ref[0:N, :]
    agg1 = (s + xp) * (dinv / deg)
    xi = lax.dot_general(agg1, w1_ref[...], (((1,), (1,)), ((), ())),
                         preferred_element_type=jnp.float32)
    xi = xi + b1_ref[...][None, :]
    xi = jnp.where(xi >= 0.0, xi, 0.01 * xi)
    mu = jnp.mean(xi, axis=0, keepdims=True)
    var = jnp.mean((xi - mu) ** 2, axis=0, keepdims=True)
    xin = (xi - mu) * lax.rsqrt(var + 1e-5)
    h2 = lax.dot_general(xin, w2_ref[...], (((1,), (1,)), ((), ())),
                         preferred_element_type=jnp.float32)
    h2_ref[...] = jnp.concatenate(
        [h2, jnp.zeros((NTRASH, D_OUT), jnp.float32)], axis=0)


def _final_body(s2_ref, cnt_ref, out_ref):
    indeg = _indeg(cnt_ref)
    inv2 = 1.0 / jnp.maximum(indeg, 1.0)
    out_ref[...] = jax.nn.sigmoid((s2_ref[0, 0:N, :] + s2_ref[1, 0:N, :]) * inv2)


def kernel(x, edge_index, W1, b1, W2):
    src = edge_index[0].astype(jnp.int32)
    dst = edge_index[1].astype(jnp.int32)
    pad_fill = N + (jnp.arange(PAD, dtype=jnp.int32) % NTRASH)
    src_p = jnp.concatenate([src, pad_fill]).reshape(ROWS, CH)
    dst_p = jnp.concatenate([dst, pad_fill]).reshape(ROWS, CH)
    zeros_d = jnp.zeros((ZR, D_IN), jnp.float32)
    zeros_c = jnp.zeros((ZR, CW), jnp.float32)
    ones_c = jnp.ones((CH, CW), jnp.float32)

    seg, cnt_call = _make_sc_calls()

    cnt = cnt_call(dst_p, zeros_c, ones_c).reshape(NC, NPAD, CW)

    xp_pad = pl.pallas_call(
        _prescale_body,
        out_shape=jax.ShapeDtypeStruct((NPAD, D_IN), jnp.float32),
    )(x, cnt)

    sp = seg(xp_pad, src_p, dst_p, zeros_d).reshape(NC, NPAD, D_IN)

    h2_pad = pl.pallas_call(
        _dense_body,
        out_shape=jax.ShapeDtypeStruct((NPAD, D_OUT), jnp.float32),
    )(sp, xp_pad, cnt, W1, b1, W2)

    s2 = seg(h2_pad, src_p, dst_p, zeros_d).reshape(NC, NPAD, D_OUT)

    out = pl.pallas_call(
        _final_body,
        out_shape=jax.ShapeDtypeStruct((N, D_OUT), jnp.float32),
    )(s2, cnt)
    return out


# speedup vs baseline: 30.0112x; 1.0079x over previous
import jax
import jax.numpy as jnp
from jax import lax
from jax.experimental import pallas as pl
from jax.experimental.pallas import tpu as pltpu
from jax.experimental.pallas import tpu_sc as plsc

N = 10000
E = 320000
D_IN = 128
D_HID = 256
D_OUT = 128

NC = 2
NS = 16
NW = NC * NS
CH = 128
NCH_W = 80
EP = NW * NCH_W * CH
PAD = EP - E
ROWS = EP // CH
NTRASH = 112
NPAD = N + NTRASH
ZR = NPAD // NS
CW = 8


HB = NCH_W // 2


def _run_window(xp_hbm, src_hbm, dst_hbm, acc_sh,
                src_v, dst_v, rows_v, gsem0, gsem1, ssem, row0):
    pltpu.sync_copy(src_hbm.at[pl.ds(row0, HB)], src_v)
    pltpu.sync_copy(dst_hbm.at[pl.ds(row0, HB)], dst_v)


    pltpu.async_copy(xp_hbm.at[src_v.at[0]], rows_v.at[0], gsem0)
    pltpu.async_copy(xp_hbm.at[src_v.at[1]], rows_v.at[1], gsem1)

    def body(g, carry):
        j = 2 * g
        pltpu.make_async_copy(xp_hbm.at[src_v.at[j]], rows_v.at[0],
                              gsem0).wait()
        pltpu.async_copy(rows_v.at[0], acc_sh.at[dst_v.at[j]], ssem,
                         add=True).wait()
        pltpu.async_copy(xp_hbm.at[src_v.at[j + 2]], rows_v.at[0], gsem0)
        pltpu.make_async_copy(xp_hbm.at[src_v.at[j + 1]], rows_v.at[1],
                              gsem1).wait()
        pltpu.async_copy(rows_v.at[1], acc_sh.at[dst_v.at[j + 1]], ssem,
                         add=True).wait()
        pltpu.async_copy(xp_hbm.at[src_v.at[j + 3]], rows_v.at[1], gsem1)
        return carry

    lax.fori_loop(0, HB // 2 - 1, body, 0)
    pltpu.make_async_copy(xp_hbm.at[src_v.at[HB - 2]], rows_v.at[0],
                          gsem0).wait()
    pltpu.async_copy(rows_v.at[0], acc_sh.at[dst_v.at[HB - 2]], ssem,
                     add=True).wait()
    pltpu.make_async_copy(xp_hbm.at[src_v.at[HB - 1]], rows_v.at[1],
                          gsem1).wait()
    pltpu.async_copy(rows_v.at[1], acc_sh.at[dst_v.at[HB - 1]], ssem,
                     add=True).wait()


def _seg_body(xp_hbm, src_hbm, dst_hbm, zeros_hbm, out_hbm,
              acc_sh, src_v, dst_v, rows_v, gsem0, gsem1, ssem):
    c = lax.axis_index("c")
    s = lax.axis_index("s")
    wid = c * NS + s
    pltpu.sync_copy(zeros_hbm, acc_sh.at[pl.ds(s * ZR, ZR)])
    plsc.subcore_barrier()
    _run_window(xp_hbm, src_hbm, dst_hbm, acc_sh,
                src_v, dst_v, rows_v, gsem0, gsem1, ssem, wid * NCH_W)
    _run_window(xp_hbm, src_hbm, dst_hbm, acc_sh,
                src_v, dst_v, rows_v, gsem0, gsem1, ssem, wid * NCH_W + HB)
    plsc.subcore_barrier()
    pltpu.sync_copy(acc_sh.at[pl.ds(s * ZR, ZR)], out_hbm.at[wid])


def _cnt_body(dst_hbm, zeros_hbm, ones_hbm, out_hbm,
              acc_sh, dst_v, ones_v, csem0, csem1):
    c = lax.axis_index("c")
    s = lax.axis_index("s")
    wid = c * NS + s
    pltpu.sync_copy(zeros_hbm, acc_sh.at[pl.ds(s * ZR, ZR)])
    pltpu.sync_copy(ones_hbm, ones_v)
    pltpu.sync_copy(dst_hbm.at[pl.ds(wid * NCH_W, NCH_W)], dst_v)
    plsc.subcore_barrier()

    def body(g, carry):
        j = 2 * g
        sd0 = pltpu.async_copy(ones_v, acc_sh.at[dst_v.at[j]], csem0,
                               add=True)
        sd1 = pltpu.async_copy(ones_v, acc_sh.at[dst_v.at[j + 1]], csem1,
                               add=True)
        sd0.wait()
        sd1.wait()
        return carry

    lax.fori_loop(0, NCH_W // 2, body, 0)
    plsc.subcore_barrier()
    pltpu.sync_copy(acc_sh.at[pl.ds(s * ZR, ZR)], out_hbm.at[wid])


def _make_sc_calls():
    mesh = plsc.VectorSubcoreMesh(core_axis_name="c", subcore_axis_name="s")
    seg = pl.kernel(
        _seg_body,
        out_type=jax.ShapeDtypeStruct((NW, ZR, D_IN), jnp.float32),
        mesh=mesh,
        scratch_types=[
            pltpu.VMEM_SHARED((NPAD, D_IN), jnp.float32),
            pltpu.VMEM((HB, CH), jnp.int32),
            pltpu.VMEM((HB, CH), jnp.int32),
            pltpu.VMEM((2, CH, D_IN), jnp.float32),
            pltpu.SemaphoreType.DMA,
            pltpu.SemaphoreType.DMA,
            pltpu.SemaphoreType.DMA,
        ],
    )
    cnt = pl.kernel(
        _cnt_body,
        out_type=jax.ShapeDtypeStruct((NW, ZR, CW), jnp.float32),
        mesh=mesh,
        scratch_types=[
            pltpu.VMEM_SHARED((NPAD, CW), jnp.float32),
            pltpu.VMEM((NCH_W, CH), jnp.int32),
            pltpu.VMEM((CH, CW), jnp.float32),
            pltpu.SemaphoreType.DMA,
            pltpu.SemaphoreType.DMA,
        ],
    )
    return seg, cnt


def _indeg(cnt_ref):
    return cnt_ref[0, 0:N, 0:1] + cnt_ref[1, 0:N, 0:1]


def _prescale_body(x_ref, cnt_ref, xp_ref):
    deg = _indeg(cnt_ref) + 1.0
    dinv = lax.rsqrt(deg)
    xp_ref[...] = jnp.concatenate(
        [x_ref[...] * dinv, jnp.zeros((NTRASH, D_IN), jnp.float32)], axis=0)


def _dense_body(sp_ref, xp_ref, cnt_ref, w1_ref, b1_ref, w2_ref, h2_ref):
    indeg = _indeg(cnt_ref)
    deg = indeg + 1.0
    dinv = lax.rsqrt(deg)
    s = sp_ref[0, 0:N, :] + sp_ref[1, 0:N, :]
    xp = xp_ref[0:N, :]
    agg1 = (s + xp) * (dinv / deg)
    xi = lax.dot_general(agg1, w1_ref[...], (((1,), (1,)), ((), ())),
                         preferred_element_type=jnp.float32)
    xi = xi + b1_ref[...][None, :]
    xi = jnp.where(xi >= 0.0, xi, 0.01 * xi)
    mu = jnp.mean(xi, axis=0, keepdims=True)
    var = jnp.mean((xi - mu) ** 2, axis=0, keepdims=True)
    xin = (xi - mu) * lax.rsqrt(var + 1e-5)
    h2 = lax.dot_general(xin, w2_ref[...], (((1,), (1,)), ((), ())),
                         preferred_element_type=jnp.float32)
    h2_ref[...] = jnp.concatenate(
        [h2, jnp.zeros((NTRASH, D_OUT), jnp.float32)], axis=0)


def _final_body(s2_ref, cnt_ref, out_ref):
    indeg = _indeg(cnt_ref)
    inv2 = 1.0 / jnp.maximum(indeg, 1.0)
    out_ref[...] = jax.nn.sigmoid((s2_ref[0, 0:N, :] + s2_ref[1, 0:N, :]) * inv2)


def kernel(x, edge_index, W1, b1, W2):
    src = edge_index[0].astype(jnp.int32)
    dst = edge_index[1].astype(jnp.int32)
    pad_fill = N + (jnp.arange(PAD, dtype=jnp.int32) % NTRASH)
    src_p = jnp.concatenate([src, pad_fill]).reshape(ROWS, CH)
    dst_p = jnp.concatenate([dst, pad_fill]).reshape(ROWS, CH)
    zeros_d = jnp.zeros((ZR, D_IN), jnp.float32)
    zeros_c = jnp.zeros((ZR, CW), jnp.float32)
    ones_c = jnp.ones((CH, CW), jnp.float32)

    seg, cnt_call = _make_sc_calls()

    cnt = cnt_call(dst_p, zeros_c, ones_c).reshape(NC, NPAD, CW)

    xp_pad = pl.pallas_call(
        _prescale_body,
        out_shape=jax.ShapeDtypeStruct((NPAD, D_IN), jnp.float32),
    )(x, cnt)

    sp = seg(xp_pad, src_p, dst_p, zeros_d).reshape(NC, NPAD, D_IN)

    h2_pad = pl.pallas_call(
        _dense_body,
        out_shape=jax.ShapeDtypeStruct((NPAD, D_OUT), jnp.float32),
    )(sp, xp_pad, cnt, W1, b1, W2)

    s2 = seg(h2_pad, src_p, dst_p, zeros_d).reshape(NC, NPAD, D_OUT)

    out = pl.pallas_call(
        _final_body,
        out_shape=jax.ShapeDtypeStruct((N, D_OUT), jnp.float32),
    )(s2, cnt)
    return out
